# baseline (device time: 42796 ns/iter reference)
import os

import jax
import jax.numpy as jnp
from jax import lax
from jax.experimental import pallas as pl
from jax.experimental.pallas import tpu as pltpu

_VARIANT = os.environ.get("KERNEL_VARIANT", "full")

N_DEV = 8
M = 768
SLAB = M // 3
N_BF = 3
N_STAGE = 3
AX_X, AX_Y, AX_Z = 1, 3, 4
PERM = ((AX_Z, AX_Y, AX_X),
        (AX_Y, AX_X, AX_Z),
        (AX_X, AX_Z, AX_Y))


def kernel(x, W1, W2):
    m, k = x.shape
    _, h_per = W1.shape
    _, n = W2.shape

    def body(x_ref, w1_ref, w2_ref, out_ref, part_ref,
             rs0_ref, rs1_ref, rs2_ref, xb_ref, w1b_ref, w2b_ref,
             send_sems, recv_sems):
        my = lax.axis_index("i")

        xb_ref[:, :] = x_ref[:, :].astype(jnp.bfloat16)
        w1b_ref[:, :] = w1_ref[:, :].astype(jnp.bfloat16)
        w2b_ref[:, :] = w2_ref[:, :].astype(jnp.bfloat16)

        def axis_bit(mask):
            if mask == AX_X:
                return jnp.bitwise_and(jnp.bitwise_xor(my, my >> 1), 1)
            if mask == AX_Y:
                return jnp.bitwise_and(my >> 1, 1)
            return jnp.bitwise_and(my >> 2, 1)

        def partner(mask):
            return jnp.bitwise_xor(my, mask)

        barrier_sem = pltpu.get_barrier_semaphore()
        for mask in (AX_X, AX_Y, AX_Z):
            pl.semaphore_signal(
                barrier_sem, inc=1,
                device_id=(partner(mask),),
                device_id_type=pl.DeviceIdType.MESH,
            )
        pl.semaphore_wait(barrier_sem, 3)

        def compute_rows(off, size):
            rows = pl.ds(off, size)
            hblk = jnp.maximum(
                jnp.dot(xb_ref[rows, :], w1b_ref[:, :],
                        preferred_element_type=jnp.float32),
                0.0,
            ).astype(jnp.bfloat16)
            return jnp.dot(hblk, w2b_ref[:, :],
                           preferred_element_type=jnp.float32)

        if _VARIANT == "compute_only":
            for b in range(N_BF):
                out_ref[pl.ds(b * SLAB, SLAB), :] = compute_rows(
                    b * SLAB, SLAB)
            return

        rs_bufs = (rs0_ref, rs1_ref, rs2_ref)

        def rs_rdma(b, s, send_off, half):
            return pltpu.make_async_remote_copy(
                src_ref=part_ref.at[b, pl.ds(send_off, half), :],
                dst_ref=rs_bufs[s].at[b],
                send_sem=send_sems.at[b * N_STAGE + s],
                recv_sem=recv_sems.at[b * N_STAGE + s],
                device_id=(partner(PERM[b][s]),),
                device_id_type=pl.DeviceIdType.MESH,
            )

        def ag_rdma(b, s, cur_off, cur_sz):
            idx = 9 + b * N_STAGE + s
            return pltpu.make_async_remote_copy(
                src_ref=out_ref.at[pl.ds(cur_off, cur_sz), :],
                dst_ref=out_ref.at[pl.ds(cur_off, cur_sz), :],
                send_sem=send_sems.at[idx],
                recv_sem=recv_sems.at[idx],
                device_id=(partner(PERM[b][2 - s]),),
                device_id_type=pl.DeviceIdType.MESH,
            )

        offs = []
        sends = []
        for b in range(N_BF):
            part_ref[b, :, :] = compute_rows(b * SLAB, SLAB)
            bkeep = axis_bit(PERM[b][0])
            half = SLAB // 2
            send_off = (1 - bkeep) * half
            d = rs_rdma(b, 0, send_off, half)
            d.start()
            sends.append(d)
            offs.append(bkeep * half)

        ag_state = [None] * N_BF
        for s in range(N_STAGE):
            half = SLAB >> (s + 1)
            for b in range(N_BF):
                rs_rdma(b, s, 0, half).wait_recv()
                koff = offs[b]
                part_ref[b, pl.ds(koff, half), :] = (
                    part_ref[b, pl.ds(koff, half), :] + rs_bufs[s][b]
                )
                if s < N_STAGE - 1:
                    nhalf = half // 2
                    bkeep = axis_bit(PERM[b][s + 1])
                    send_off = koff + (1 - bkeep) * nhalf
                    d = rs_rdma(b, s + 1, send_off, nhalf)
                    d.start()
                    sends.append(d)
                    offs[b] = koff + bkeep * nhalf
                else:
                    cur_off = b * SLAB + koff
                    out_ref[pl.ds(cur_off, half), :] = part_ref[
                        b, pl.ds(koff, half), :]
                    d = ag_rdma(b, 0, cur_off, half)
                    d.start()
                    sends.append(d)
                    bkeep = axis_bit(PERM[b][2])
                    ag_state[b] = (cur_off - bkeep * half, half)

        for s in range(N_STAGE):
            for b in range(N_BF):
                par_off, sz = ag_state[b]
                ag_rdma(b, s, par_off, sz).wait_recv()
                if s < N_STAGE - 1:
                    d = ag_rdma(b, s + 1, par_off, 2 * sz)
                    d.start()
                    sends.append(d)
                    bkeep = axis_bit(PERM[b][2 - (s + 1)])
                    ag_state[b] = (par_off - bkeep * 2 * sz, 2 * sz)

        for d in sends:
            d.wait_send()

    return pl.pallas_call(
        body,
        out_shape=jax.ShapeDtypeStruct((m, n), jnp.float32),
        in_specs=[
            pl.BlockSpec(memory_space=pltpu.VMEM),
            pl.BlockSpec(memory_space=pltpu.VMEM),
            pl.BlockSpec(memory_space=pltpu.VMEM),
        ],
        out_specs=pl.BlockSpec(memory_space=pltpu.VMEM),
        scratch_shapes=[
            pltpu.VMEM((N_BF, SLAB, n), jnp.float32),
            pltpu.VMEM((N_BF, SLAB // 2, n), jnp.float32),
            pltpu.VMEM((N_BF, SLAB // 4, n), jnp.float32),
            pltpu.VMEM((N_BF, SLAB // 8, n), jnp.float32),
            pltpu.VMEM((m, k), jnp.bfloat16),
            pltpu.VMEM((k, h_per), jnp.bfloat16),
            pltpu.VMEM((h_per, n), jnp.bfloat16),
            pltpu.SemaphoreType.DMA((18,)),
            pltpu.SemaphoreType.DMA((18,)),
        ],
        compiler_params=pltpu.CompilerParams(
            collective_id=0,
            vmem_limit_bytes=100 * 1024 * 1024,
        ),
    )(x, W1, W2)


# device time: 34589 ns/iter; 1.2373x vs baseline; 1.2373x over previous
import os

import jax
import jax.numpy as jnp
from jax import lax
from jax.experimental import pallas as pl
from jax.experimental.pallas import tpu as pltpu

_VARIANT = os.environ.get("KERNEL_VARIANT", "full")

N_DEV = 8
M = 768
N_BF = 3
N_STAGE = 3
SLABS = (384, 256, 128)
BASES = (0, 384, 640)
PBASE = (0, 192, 320)
AX_X, AX_Y, AX_Z = 1, 3, 4
PERM = ((AX_Z, AX_Y, AX_X),
        (AX_Y, AX_X, AX_Z),
        (AX_X, AX_Z, AX_Y))


def kernel(x, W1, W2):
    m, k = x.shape
    _, h_per = W1.shape
    _, n = W2.shape

    def body(x_ref, w1_ref, w2_ref, out_ref, part_ref, res_ref,
             rs0_ref, rs1_ref, rs2_ref, sb0_ref, sb1_ref, sb2_ref,
             xb_ref, w1b_ref, w2b_ref, send_sems, recv_sems):
        my = lax.axis_index("i")

        xb_ref[:, :] = x_ref[:, :].astype(jnp.bfloat16)
        w1b_ref[:, :] = w1_ref[:, :].astype(jnp.bfloat16)
        w2b_ref[:, :] = w2_ref[:, :].astype(jnp.bfloat16)

        bits = {
            AX_X: jnp.bitwise_and(jnp.bitwise_xor(my, my >> 1), 1),
            AX_Y: jnp.bitwise_and(my >> 1, 1),
            AX_Z: jnp.bitwise_and(my >> 2, 1),
        }

        def partner(mask):
            return jnp.bitwise_xor(my, mask)

        barrier_sem = pltpu.get_barrier_semaphore()
        for mask in (AX_X, AX_Y, AX_Z):
            pl.semaphore_signal(
                barrier_sem, inc=1,
                device_id=(partner(mask),),
                device_id_type=pl.DeviceIdType.MESH,
            )
        pl.semaphore_wait(barrier_sem, 3)

        def compute_rows(off, size):
            rows = pl.ds(off, size)
            hblk = jnp.maximum(
                jnp.dot(xb_ref[rows, :], w1b_ref[:, :],
                        preferred_element_type=jnp.float32),
                0.0,
            ).astype(jnp.bfloat16)
            return jnp.dot(hblk, w2b_ref[:, :],
                           preferred_element_type=jnp.float32)

        if _VARIANT == "compute_only":
            for b in range(N_BF):
                out_ref[pl.ds(BASES[b], SLABS[b]), :] = compute_rows(
                    BASES[b], SLABS[b])
            return

        rs_bufs = (rs0_ref, rs1_ref, rs2_ref)
        sbufs = (sb0_ref, sb1_ref, sb2_ref)

        def rs_rdma(b, s, size):
            return pltpu.make_async_remote_copy(
                src_ref=sbufs[s].at[b, pl.ds(0, size), :],
                dst_ref=rs_bufs[s].at[b, pl.ds(0, size), :],
                send_sem=send_sems.at[b * N_STAGE + s],
                recv_sem=recv_sems.at[b * N_STAGE + s],
                device_id=(partner(PERM[b][s]),),
                device_id_type=pl.DeviceIdType.MESH,
            )

        def ag_rdma(b, s, off, size):
            idx = 9 + b * N_STAGE + s
            return pltpu.make_async_remote_copy(
                src_ref=res_ref.at[pl.ds(off, size), :],
                dst_ref=res_ref.at[pl.ds(off, size), :],
                send_sem=send_sems.at[idx],
                recv_sem=recv_sems.at[idx],
                device_id=(partner(PERM[b][2 - s]),),
                device_id_type=pl.DeviceIdType.MESH,
            )

        sends = []
        koff = [None] * N_BF
        goff = [None] * N_BF

        for b in range(N_BF):
            half = SLABS[b] // 2
            bk = bits[PERM[b][0]]
            sh_off = BASES[b] + (1 - bk) * half
            sb0_ref[b, pl.ds(0, half), :] = compute_rows(
                sh_off, half).astype(jnp.bfloat16)
            d = rs_rdma(b, 0, half)
            d.start()
            sends.append(d)
            koff[b] = PBASE[b]
            goff[b] = BASES[b] + bk * half

        for b in range(N_BF):
            half = SLABS[b] // 2
            part_ref[pl.ds(PBASE[b], half), :] = compute_rows(
                goff[b], half)
            rs_rdma(b, 0, half).wait_recv()
            part_ref[pl.ds(koff[b], half), :] = (
                part_ref[pl.ds(koff[b], half), :]
                + rs0_ref[b, pl.ds(0, half), :].astype(jnp.float32)
            )
            nh = half // 2
            bk = bits[PERM[b][1]]
            sb1_ref[b, pl.ds(0, nh), :] = part_ref[
                pl.ds(koff[b] + (1 - bk) * nh, nh), :].astype(jnp.bfloat16)
            d = rs_rdma(b, 1, nh)
            d.start()
            sends.append(d)
            koff[b] = koff[b] + bk * nh
            goff[b] = goff[b] + bk * nh

        ag_state = [None] * N_BF
        for s in (1, 2):
            for b in range(N_BF):
                h = SLABS[b] >> (s + 1)
                rs_rdma(b, s, h).wait_recv()
                part_ref[pl.ds(koff[b], h), :] = (
                    part_ref[pl.ds(koff[b], h), :]
                    + rs_bufs[s][b, pl.ds(0, h), :].astype(jnp.float32)
                )
                if s == 1:
                    nh = h // 2
                    bk = bits[PERM[b][2]]
                    sb2_ref[b, pl.ds(0, nh), :] = part_ref[
                        pl.ds(koff[b] + (1 - bk) * nh, nh), :
                    ].astype(jnp.bfloat16)
                    d = rs_rdma(b, 2, nh)
                    d.start()
                    sends.append(d)
                    koff[b] = koff[b] + bk * nh
                    goff[b] = goff[b] + bk * nh
                else:
                    res_ref[pl.ds(goff[b], h), :] = part_ref[
                        pl.ds(koff[b], h), :].astype(jnp.bfloat16)
                    d = ag_rdma(b, 0, goff[b], h)
                    d.start()
                    sends.append(d)
                    bk = bits[PERM[b][2]]
                    ag_state[b] = (goff[b] - bk * h, h)

        for s in range(N_STAGE):
            for b in range(N_BF):
                par_off, sz = ag_state[b]
                ag_rdma(b, s, par_off, sz).wait_recv()
                if s < N_STAGE - 1:
                    d = ag_rdma(b, s + 1, par_off, 2 * sz)
                    d.start()
                    sends.append(d)
                    bk = bits[PERM[b][2 - (s + 1)]]
                    ag_state[b] = (par_off - bk * 2 * sz, 2 * sz)

        out_ref[:, :] = res_ref[:, :].astype(jnp.float32)

        for d in sends:
            d.wait_send()

    return pl.pallas_call(
        body,
        out_shape=jax.ShapeDtypeStruct((m, n), jnp.float32),
        in_specs=[
            pl.BlockSpec(memory_space=pltpu.VMEM),
            pl.BlockSpec(memory_space=pltpu.VMEM),
            pl.BlockSpec(memory_space=pltpu.VMEM),
        ],
        out_specs=pl.BlockSpec(memory_space=pltpu.VMEM),
        scratch_shapes=[
            pltpu.VMEM((M // 2, n), jnp.float32),
            pltpu.VMEM((M, n), jnp.bfloat16),
            pltpu.VMEM((N_BF, SLABS[0] // 2, n), jnp.bfloat16),
            pltpu.VMEM((N_BF, SLABS[0] // 4, n), jnp.bfloat16),
            pltpu.VMEM((N_BF, SLABS[0] // 8, n), jnp.bfloat16),
            pltpu.VMEM((N_BF, SLABS[0] // 2, n), jnp.bfloat16),
            pltpu.VMEM((N_BF, SLABS[0] // 4, n), jnp.bfloat16),
            pltpu.VMEM((N_BF, SLABS[0] // 8, n), jnp.bfloat16),
            pltpu.VMEM((m, k), jnp.bfloat16),
            pltpu.VMEM((k, h_per), jnp.bfloat16),
            pltpu.VMEM((h_per, n), jnp.bfloat16),
            pltpu.SemaphoreType.DMA((18,)),
            pltpu.SemaphoreType.DMA((18,)),
        ],
        compiler_params=pltpu.CompilerParams(
            collective_id=0,
            vmem_limit_bytes=100 * 1024 * 1024,
        ),
    )(x, W1, W2)
